# baseline (device time: 17020 ns/iter reference)
import os

import jax
import jax.numpy as jnp
from jax import lax
from jax.experimental import pallas as pl
from jax.experimental.pallas import tpu as pltpu

N_DEV = 16
_MODE = os.environ.get("KMODE", "full")


def kernel(x, W, labels):
    T, D = x.shape
    V_LOC = W.shape[1]

    def body(
        x_ref,
        w_hbm_ref,
        lab_ref,
        out_ref,
        w_ref,
        comm_ref,
        w_sem,
        send_sems,
        recv_sems,
    ):
        my = lax.axis_index("i")

        w_cp = pltpu.make_async_copy(w_hbm_ref, w_ref, w_sem)
        w_cp.start()

        if _MODE != "compute":
            barrier_sem = pltpu.get_barrier_semaphore()
            for d in range(1, N_DEV):
                pl.semaphore_signal(
                    barrier_sem,
                    inc=1,
                    device_id=(lax.rem(my + d, N_DEV),),
                    device_id_type=pl.DeviceIdType.MESH,
                )

        w_cp.wait()

        logits = jnp.dot(
            x_ref[:, :].astype(jnp.bfloat16),
            w_ref[:, :].astype(jnp.bfloat16),
            preferred_element_type=jnp.float32,
        ).astype(jnp.bfloat16)
        m = jnp.max(logits, axis=1)
        s = jnp.sum(
            jnp.exp(logits - m[:, None]), axis=1, dtype=jnp.float32
        )
        local_tgt = lab_ref[:] - my * V_LOC
        col = lax.broadcasted_iota(jnp.int32, (T, V_LOC), 1)
        lab = jnp.sum(
            jnp.where(col == local_tgt[:, None], logits, jnp.bfloat16(0.0)),
            axis=1,
            dtype=jnp.float32,
        )

        comm_ref[0, 0, :] = m.astype(jnp.float32)
        comm_ref[0, 1, :] = s
        comm_ref[0, 2, :] = lab

        if _MODE != "compute":
            pl.semaphore_wait(barrier_sem, N_DEV - 1)

            rdmas = []
            for d in range(1, N_DEV):
                rdma = pltpu.make_async_remote_copy(
                    src_ref=comm_ref.at[0],
                    dst_ref=comm_ref.at[d],
                    send_sem=send_sems.at[d],
                    recv_sem=recv_sems.at[d],
                    device_id=(lax.rem(my + d, N_DEV),),
                    device_id_type=pl.DeviceIdType.MESH,
                )
                rdma.start()
                rdmas.append(rdma)
            for rdma in rdmas:
                rdma.wait()

        allm = comm_ref[:, 0, :]
        alls = comm_ref[:, 1, :]
        alllab = comm_ref[:, 2, :]
        M = jnp.max(allm, axis=0)
        Z = jnp.sum(alls * jnp.exp(allm - M[None, :]), axis=0)
        lab_tot = jnp.sum(alllab, axis=0)
        out_ref[:] = M + jnp.log(Z) - lab_tot

    return pl.pallas_call(
        body,
        out_shape=jax.ShapeDtypeStruct((T,), jnp.float32),
        in_specs=[
            pl.BlockSpec(memory_space=pltpu.VMEM),
            pl.BlockSpec(memory_space=pl.ANY),
            pl.BlockSpec(memory_space=pltpu.VMEM),
        ],
        out_specs=pl.BlockSpec(memory_space=pltpu.VMEM),
        scratch_shapes=[
            pltpu.VMEM((D, V_LOC), jnp.float32),
            pltpu.VMEM((N_DEV, 3, T), jnp.float32),
            pltpu.SemaphoreType.DMA(()),
            pltpu.SemaphoreType.DMA((N_DEV,)),
            pltpu.SemaphoreType.DMA((N_DEV,)),
        ],
        **(
            {}
            if _MODE == "compute"
            else dict(compiler_params=pltpu.CompilerParams(collective_id=0))
        ),
    )(x, W, labels)


# device time: 16283 ns/iter; 1.0453x vs baseline; 1.0453x over previous
import os

import jax
import jax.numpy as jnp
from jax import lax
from jax.experimental import pallas as pl
from jax.experimental.pallas import tpu as pltpu

N_DEV = 16
_MODE = os.environ.get("KMODE", "full")


def kernel(x, W, labels):
    T, D = x.shape
    V_LOC = W.shape[1]
    W = pltpu.with_memory_space_constraint(W, pltpu.MemorySpace.HBM)

    def body(
        x_ref,
        w_hbm_ref,
        lab_ref,
        out_ref,
        w_ref,
        comm_ref,
        w_sem,
        send_sems,
        recv_sems,
    ):
        my = lax.axis_index("i")

        w_cp = pltpu.make_async_copy(w_hbm_ref, w_ref, w_sem)
        w_cp.start()

        if _MODE != "compute":
            barrier_sem = pltpu.get_barrier_semaphore()
            for d in range(1, N_DEV):
                pl.semaphore_signal(
                    barrier_sem,
                    inc=1,
                    device_id=(lax.rem(my + d, N_DEV),),
                    device_id_type=pl.DeviceIdType.MESH,
                )

        w_cp.wait()

        logits = jnp.dot(
            x_ref[:, :].astype(jnp.bfloat16),
            w_ref[:, :].astype(jnp.bfloat16),
            preferred_element_type=jnp.float32,
        ).astype(jnp.bfloat16)
        m = jnp.max(logits, axis=1)
        s = jnp.sum(
            jnp.exp(logits - m[:, None]), axis=1, dtype=jnp.float32
        )
        local_tgt = lab_ref[:] - my * V_LOC
        col = lax.broadcasted_iota(jnp.int32, (T, V_LOC), 1)
        lab = jnp.sum(
            jnp.where(col == local_tgt[:, None], logits, jnp.bfloat16(0.0)),
            axis=1,
            dtype=jnp.float32,
        )

        comm_ref[0, 0, :] = m.astype(jnp.float32)
        comm_ref[0, 1, :] = s
        comm_ref[0, 2, :] = lab

        if _MODE != "compute":
            pl.semaphore_wait(barrier_sem, N_DEV - 1)

            rdmas = []
            for d in range(1, N_DEV):
                rdma = pltpu.make_async_remote_copy(
                    src_ref=comm_ref.at[0],
                    dst_ref=comm_ref.at[d],
                    send_sem=send_sems.at[d],
                    recv_sem=recv_sems.at[d],
                    device_id=(lax.rem(my + d, N_DEV),),
                    device_id_type=pl.DeviceIdType.MESH,
                )
                rdma.start()
                rdmas.append(rdma)
            for rdma in rdmas:
                rdma.wait()

        allm = comm_ref[:, 0, :]
        alls = comm_ref[:, 1, :]
        alllab = comm_ref[:, 2, :]
        M = jnp.max(allm, axis=0)
        Z = jnp.sum(alls * jnp.exp(allm - M[None, :]), axis=0)
        lab_tot = jnp.sum(alllab, axis=0)
        out_ref[:] = M + jnp.log(Z) - lab_tot

    return pl.pallas_call(
        body,
        out_shape=jax.ShapeDtypeStruct((T,), jnp.float32),
        in_specs=[
            pl.BlockSpec(memory_space=pltpu.VMEM),
            pl.BlockSpec(memory_space=pl.ANY),
            pl.BlockSpec(memory_space=pltpu.VMEM),
        ],
        out_specs=pl.BlockSpec(memory_space=pltpu.VMEM),
        scratch_shapes=[
            pltpu.VMEM((D, V_LOC), jnp.float32),
            pltpu.VMEM((N_DEV, 3, T), jnp.float32),
            pltpu.SemaphoreType.DMA(()),
            pltpu.SemaphoreType.DMA((N_DEV,)),
            pltpu.SemaphoreType.DMA((N_DEV,)),
        ],
        **(
            {}
            if _MODE == "compute"
            else dict(compiler_params=pltpu.CompilerParams(collective_id=0))
        ),
    )(x, W, labels)


# device time: 7068 ns/iter; 2.4080x vs baseline; 2.3038x over previous
import os

import jax
import jax.numpy as jnp
from jax import lax
from jax.experimental import pallas as pl
from jax.experimental.pallas import tpu as pltpu

N_DEV = 16
N_CHUNK = 4
_MODE = os.environ.get("KMODE", "full")


def kernel(x, W, labels):
    T, D = x.shape
    V_LOC = W.shape[1]
    CH = V_LOC // N_CHUNK
    x = pltpu.with_memory_space_constraint(x, pltpu.MemorySpace.HBM)
    W = pltpu.with_memory_space_constraint(W, pltpu.MemorySpace.HBM)
    labels = pltpu.with_memory_space_constraint(labels, pltpu.MemorySpace.HBM)

    def body(
        x_hbm,
        w_hbm,
        lab_hbm,
        out_ref,
        x_ref,
        w_ref,
        lab_ref,
        comm_ref,
        in_sems,
        w_sems,
        send_sems,
        recv_sems,
    ):
        my = lax.axis_index("i")

        x_cp = pltpu.make_async_copy(x_hbm, x_ref, in_sems.at[0])
        lab_cp = pltpu.make_async_copy(lab_hbm, lab_ref, in_sems.at[1])
        x_cp.start()
        lab_cp.start()
        w_cps = []
        for c in range(N_CHUNK):
            cols = pl.ds(c * CH, CH)
            cp = pltpu.make_async_copy(
                w_hbm.at[:, cols], w_ref.at[:, cols], w_sems.at[c]
            )
            cp.start()
            w_cps.append(cp)

        if _MODE != "compute":
            barrier_sem = pltpu.get_barrier_semaphore()
            for d in range(1, N_DEV):
                pl.semaphore_signal(
                    barrier_sem,
                    inc=1,
                    device_id=(lax.rem(my + d, N_DEV),),
                    device_id_type=pl.DeviceIdType.MESH,
                )

        x_cp.wait()
        xb = x_ref[:, :].astype(jnp.bfloat16)
        lab_cp.wait()
        local_tgt = lab_ref[:] - my * V_LOC
        col = lax.broadcasted_iota(jnp.int32, (T, CH), 1)

        ms, ss, labs = [], [], []
        for c in range(N_CHUNK):
            w_cps[c].wait()
            logits = jnp.dot(
                xb,
                w_ref[:, pl.ds(c * CH, CH)].astype(jnp.bfloat16),
                preferred_element_type=jnp.float32,
            ).astype(jnp.bfloat16)
            m_c = jnp.max(logits, axis=1)
            s_c = jnp.sum(
                jnp.exp(logits - m_c[:, None]), axis=1, dtype=jnp.float32
            )
            tgt_c = local_tgt - c * CH
            lab_c = jnp.sum(
                jnp.where(col == tgt_c[:, None], logits, jnp.bfloat16(0.0)),
                axis=1,
                dtype=jnp.float32,
            )
            ms.append(m_c.astype(jnp.float32))
            ss.append(s_c)
            labs.append(lab_c)

        cm = jnp.stack(ms)
        m = jnp.max(cm, axis=0)
        s = jnp.sum(jnp.stack(ss) * jnp.exp(cm - m[None, :]), axis=0)
        lab = jnp.sum(jnp.stack(labs), axis=0)

        comm_ref[0, 0, :] = m
        comm_ref[0, 1, :] = s
        comm_ref[0, 2, :] = lab

        if _MODE != "compute":
            pl.semaphore_wait(barrier_sem, N_DEV - 1)

            rdmas = []
            for d in range(1, N_DEV):
                rdma = pltpu.make_async_remote_copy(
                    src_ref=comm_ref.at[0],
                    dst_ref=comm_ref.at[d],
                    send_sem=send_sems.at[d],
                    recv_sem=recv_sems.at[d],
                    device_id=(lax.rem(my + d, N_DEV),),
                    device_id_type=pl.DeviceIdType.MESH,
                )
                rdma.start()
                rdmas.append(rdma)
            for rdma in rdmas:
                rdma.wait()

        allm = comm_ref[:, 0, :]
        alls = comm_ref[:, 1, :]
        alllab = comm_ref[:, 2, :]
        M = jnp.max(allm, axis=0)
        Z = jnp.sum(alls * jnp.exp(allm - M[None, :]), axis=0)
        lab_tot = jnp.sum(alllab, axis=0)
        out_ref[:] = M + jnp.log(Z) - lab_tot

    return pl.pallas_call(
        body,
        out_shape=jax.ShapeDtypeStruct((T,), jnp.float32),
        in_specs=[
            pl.BlockSpec(memory_space=pl.ANY),
            pl.BlockSpec(memory_space=pl.ANY),
            pl.BlockSpec(memory_space=pl.ANY),
        ],
        out_specs=pl.BlockSpec(memory_space=pltpu.VMEM),
        scratch_shapes=[
            pltpu.VMEM((T, D), jnp.float32),
            pltpu.VMEM((D, V_LOC), jnp.float32),
            pltpu.VMEM((T,), jnp.int32),
            pltpu.VMEM((N_DEV, 3, T), jnp.float32),
            pltpu.SemaphoreType.DMA((2,)),
            pltpu.SemaphoreType.DMA((N_CHUNK,)),
            pltpu.SemaphoreType.DMA((N_DEV,)),
            pltpu.SemaphoreType.DMA((N_DEV,)),
        ],
        **(
            {}
            if _MODE == "compute"
            else dict(compiler_params=pltpu.CompilerParams(collective_id=0))
        ),
    )(x, W, labels)
